# baseline (speedup 1.0000x reference)
import functools

import jax
import jax.numpy as jnp
from jax import lax
from jax.experimental import pallas as pl
from jax.experimental.pallas import tpu as pltpu
from jax.experimental.pallas import tpu_sc as plsc

D = 64
DP = 128
NW = 32
CHUNK = 256


def _gather_kernel(n_chunks, table_hbm, idx_hbm, out_hbm, idx_v, rows_v,
                   gsem0, gsem1):
    wid = lax.axis_index("s") * 2 + lax.axis_index("c")
    n_idx = n_chunks * CHUNK
    base_row = wid * n_idx

    pltpu.sync_copy(idx_hbm.at[pl.ds(base_row, n_idx)], idx_v)

    gsems = (gsem0, gsem1)

    for b in range(2):
        pltpu.async_copy(
            table_hbm.at[idx_v.at[pl.ds(b * CHUNK, CHUNK)]], rows_v.at[b],
            gsems[b])

    def body(c, _):
        for b in range(2):
            cc = 2 * c + b
            pltpu.make_async_copy(
                table_hbm.at[idx_v.at[pl.ds(cc * CHUNK, CHUNK)]],
                rows_v.at[b], gsems[b]).wait()
            pltpu.sync_copy(rows_v.at[b],
                            out_hbm.at[pl.ds(base_row + cc * CHUNK, CHUNK)])
            pltpu.async_copy(
                table_hbm.at[idx_v.at[pl.ds((cc + 2) * CHUNK, CHUNK)]],
                rows_v.at[b], gsems[b])
        return _

    lax.fori_loop(0, n_chunks // 2 - 1, body, 0, unroll=False)

    for b in range(2):
        cc = n_chunks - 2 + b
        pltpu.make_async_copy(
            table_hbm.at[idx_v.at[pl.ds(cc * CHUNK, CHUNK)]],
            rows_v.at[b], gsems[b]).wait()
        pltpu.sync_copy(rows_v.at[b],
                        out_hbm.at[pl.ds(base_row + cc * CHUNK, CHUNK)])


def kernel(table, encoded_captions):
    B, S = encoded_captions.shape
    N = B * S
    assert N % (NW * CHUNK * 2) == 0
    n_chunks = N // (NW * CHUNK)
    idx = encoded_captions.reshape(N).astype(jnp.int32)
    tab = lax.pad(table, jnp.float32(0), ((0, 0, 0), (0, DP - D, 0)))

    mesh = plsc.VectorSubcoreMesh(core_axis_name="c", subcore_axis_name="s")

    run = functools.partial(
        pl.kernel,
        out_type=jax.ShapeDtypeStruct((N, DP), jnp.float32),
        mesh=mesh,
        compiler_params=pltpu.CompilerParams(use_tc_tiling_on_sc=True),
        scratch_types=[
            pltpu.VMEM((N // NW,), jnp.int32),
            pltpu.VMEM((2, CHUNK, DP), jnp.float32),
            pltpu.SemaphoreType.DMA,
            pltpu.SemaphoreType.DMA,
        ],
    )(functools.partial(_gather_kernel, n_chunks))

    out = run(tab, idx)
    return out[:, :D].reshape(B, S, D)
